# baseline (speedup 1.0000x reference)
import numpy as np
import jax
import jax.numpy as jnp
from jax.experimental import pallas as pl
from jax.experimental.pallas import tpu as pltpu

N = 10000
NFEAT = 128
NCLASSES = 16

MA = 400
NJ = N // MA
W4 = 4096
W8 = 8192
W1K = 1024
WT = 384
DIAG = (NJ - 1) * MA


def _phasea_kernel(adj_ref, x_ref, w1_ref, b1_ref, w2_ref, b2_ref,
                   s2_ref, opart_ref, rhs_ref):
    i = pl.program_id(0)
    j = NJ - 1 - i

    @pl.when(i == 0)
    def _init():
        rhs_ref[:, :NFEAT] = jnp.dot(x_ref[...], w1_ref[...],
                                     preferred_element_type=jnp.float32)
        rhs_ref[:, NFEAT:] = jnp.zeros((N, NFEAT), jnp.float32)

    both = jnp.dot(adj_ref[...], rhs_ref[...],
                   preferred_element_type=jnp.float32)
    feat = jnp.maximum(both[:, :NFEAT] + b1_ref[...], 0.0)
    s2j = jnp.dot(feat, w2_ref[...], preferred_element_type=jnp.float32)
    rhs_ref[pl.ds(j * MA, MA), NFEAT:] = s2j
    s2_ref[...] = s2j
    base = both[:, NFEAT:NFEAT + NCLASSES] + b2_ref[...]

    @pl.when(j != NJ - 1)
    def _plain():
        opart_ref[pl.ds(j * MA, MA), :] = base

    @pl.when(j == NJ - 1)
    def _with_diag():
        opart_ref[pl.ds(j * MA, MA), :] = base + jnp.dot(
            adj_ref[:, DIAG:], s2j,
            preferred_element_type=jnp.float32)[:, :NCLASSES]


def _schedule_b():
    steps = []
    for j in range(NJ):
        need = MA * (j + 1)
        row = [0 if need <= W4 else 1]
        if need > W8:
            row.append(2)
        if need > W8 + W1K:
            row.append(3)
        steps.extend((j, m) for m in row)
    jm = np.array([s[0] for s in steps], np.int32)
    md = np.array([s[1] for s in steps], np.int32)
    plans = []
    for mode, first_row in ((0, 0), (1, 10), (2, 20), (3, 23)):
        cur = first_row
        plan = []
        for j, m in steps:
            if m == mode:
                cur = j
            plan.append(cur)
        plans.append(np.array(plan, np.int32))
    return jm, md, plans[0], plans[1], plans[2], plans[3]


_JM, _MD, _J4, _J8, _J1, _JT = _schedule_b()
TOTAL_B = len(_JM)


def _phaseb_kernel(jm_ref, md_ref, j4_ref, j8_ref, j1_ref, jt_ref,
                   adj4_ref, adj8_ref, adj1_ref, adjt_ref, s2_ref, opart_ref,
                   out_ref, s2r_ref):
    t = pl.program_id(0)
    j = jm_ref[t]
    md = md_ref[t]

    @pl.when(t == 0)
    def _init():
        s2r_ref[...] = jnp.zeros_like(s2r_ref)

    @pl.when(md <= 1)
    def _fill():
        base = j * MA
        s2r_ref[pl.ds(base, MA), :] = s2_ref[pl.ds(base, MA), :]

    rows = pl.ds(j * MA, MA)

    @pl.when(md == 0)
    def _m0():
        out_ref[rows, :] = opart_ref[...] + jnp.dot(
            adj4_ref[...], s2r_ref[pl.ds(0, W4), :],
            preferred_element_type=jnp.float32)[:, :NCLASSES]

    @pl.when(md == 1)
    def _m1():
        out_ref[rows, :] = opart_ref[...] + jnp.dot(
            adj8_ref[...], s2r_ref[pl.ds(0, W8), :],
            preferred_element_type=jnp.float32)[:, :NCLASSES]

    @pl.when(md == 2)
    def _m2():
        out_ref[rows, :] += jnp.dot(
            adj1_ref[...], s2r_ref[pl.ds(W8, W1K), :],
            preferred_element_type=jnp.float32)[:, :NCLASSES]

    @pl.when(md == 3)
    def _m3():
        out_ref[rows, :] += jnp.dot(
            adjt_ref[...], s2r_ref[pl.ds(W8 + W1K, WT), :],
            preferred_element_type=jnp.float32)[:, :NCLASSES]


@jax.jit
def kernel(x, adj, W1, b1, W2, b2):
    b1r = b1.reshape(1, NFEAT)
    b2r = b2.reshape(1, NCLASSES)
    w2p = jnp.pad(W2, ((0, 0), (0, NFEAT - NCLASSES)))

    s2, opart = pl.pallas_call(
        _phasea_kernel,
        grid=(NJ,),
        in_specs=[
            pl.BlockSpec((MA, N), lambda i: (NJ - 1 - i, 0)),
            pl.BlockSpec((N, NFEAT), lambda i: (0, 0)),
            pl.BlockSpec((NFEAT, NFEAT), lambda i: (0, 0)),
            pl.BlockSpec((1, NFEAT), lambda i: (0, 0)),
            pl.BlockSpec((NFEAT, NFEAT), lambda i: (0, 0)),
            pl.BlockSpec((1, NCLASSES), lambda i: (0, 0)),
        ],
        out_specs=[
            pl.BlockSpec((MA, NFEAT), lambda i: (NJ - 1 - i, 0)),
            pl.BlockSpec((N, NCLASSES), lambda i: (0, 0)),
        ],
        out_shape=[
            jax.ShapeDtypeStruct((N, NFEAT), jnp.float32),
            jax.ShapeDtypeStruct((N, NCLASSES), jnp.float32),
        ],
        scratch_shapes=[pltpu.VMEM((N, 2 * NFEAT), jnp.float32)],
        compiler_params=pltpu.CompilerParams(
            dimension_semantics=("arbitrary",)),
    )(adj, x, W1, b1r, w2p, b2r)

    grid_spec = pltpu.PrefetchScalarGridSpec(
        num_scalar_prefetch=6,
        grid=(TOTAL_B,),
        in_specs=[
            pl.BlockSpec((MA, W4),
                         lambda t, jm, md, j4, j8, j1, jt: (j4[t], 0)),
            pl.BlockSpec((MA, W8),
                         lambda t, jm, md, j4, j8, j1, jt: (j8[t], 0)),
            pl.BlockSpec((MA, W1K),
                         lambda t, jm, md, j4, j8, j1, jt: (j1[t], W8 // W1K)),
            pl.BlockSpec((MA, WT),
                         lambda t, jm, md, j4, j8, j1, jt:
                         (jt[t], (W8 + W1K) // WT)),
            pl.BlockSpec((N, NFEAT),
                         lambda t, jm, md, j4, j8, j1, jt: (0, 0)),
            pl.BlockSpec((MA, NCLASSES),
                         lambda t, jm, md, j4, j8, j1, jt: (jm[t], 0)),
        ],
        out_specs=pl.BlockSpec((N, NCLASSES),
                               lambda t, jm, md, j4, j8, j1, jt: (0, 0)),
        scratch_shapes=[
            pltpu.VMEM((N, NFEAT), jnp.float32),
        ],
    )

    logit = pl.pallas_call(
        _phaseb_kernel,
        grid_spec=grid_spec,
        out_shape=jax.ShapeDtypeStruct((N, NCLASSES), jnp.float32),
        compiler_params=pltpu.CompilerParams(
            dimension_semantics=("arbitrary",)),
    )(jnp.asarray(_JM), jnp.asarray(_MD), jnp.asarray(_J4), jnp.asarray(_J8),
      jnp.asarray(_J1), jnp.asarray(_JT), adj, adj, adj, adj, s2, opart)

    return logit
